# baseline (device time: 352881 ns/iter reference)
import jax
import jax.numpy as jnp
from jax import lax
from jax.experimental import pallas as pl
from jax.experimental.pallas import tpu as pltpu

N_DEV = 16
N_HOPS = N_DEV - 1
N_LAYERS = 3


def kernel(x, Win0, Wout0, Win1, Wout1, Win2, Wout2):
    b, d = x.shape

    def body(x_ref, win0_ref, wout0_ref, win1_ref, wout1_ref, win2_ref,
             wout2_ref, out_ref, comm_ref, send_sems, recv_sems, credit_sem):
        my = lax.axis_index("i")
        left = lax.rem(my + N_DEV - 1, N_DEV)
        right = lax.rem(my + 1, N_DEV)

        barrier_sem = pltpu.get_barrier_semaphore()
        for nbr in (left, right):
            pl.semaphore_signal(
                barrier_sem, inc=1,
                device_id=(nbr,), device_id_type=pl.DeviceIdType.MESH,
            )
        pl.semaphore_wait(barrier_sem, 2)

        wins = (win0_ref, win1_ref, win2_ref)
        wouts = (wout0_ref, wout1_ref, wout2_ref)

        x_cur = x_ref[...]
        for layer in range(N_LAYERS):
            h = jnp.maximum(
                jnp.dot(x_cur, wins[layer][...],
                        preferred_element_type=jnp.float32),
                0.0,
            )
            partial = jnp.dot(h, wouts[layer][...],
                              preferred_element_type=jnp.float32)

            if layer > 0:
                pl.semaphore_wait(credit_sem, 1)

            comm_ref[0] = partial
            acc = partial
            for hop in range(N_HOPS):
                rdma = pltpu.make_async_remote_copy(
                    src_ref=comm_ref.at[hop],
                    dst_ref=comm_ref.at[hop + 1],
                    send_sem=send_sems.at[hop],
                    recv_sem=recv_sems.at[hop],
                    device_id=(right,),
                    device_id_type=pl.DeviceIdType.MESH,
                )
                rdma.start()
                rdma.wait()
                acc = acc + comm_ref[hop + 1]

            pl.semaphore_signal(
                credit_sem, inc=1,
                device_id=(left,), device_id_type=pl.DeviceIdType.MESH,
            )
            x_cur = acc

        pl.semaphore_wait(credit_sem, 1)
        out_ref[...] = x_cur

    return pl.pallas_call(
        body,
        out_shape=jax.ShapeDtypeStruct((b, d), jnp.float32),
        in_specs=[pl.BlockSpec(memory_space=pltpu.VMEM)] * 7,
        out_specs=pl.BlockSpec(memory_space=pltpu.VMEM),
        scratch_shapes=[
            pltpu.VMEM((N_DEV, b, d), jnp.float32),
            pltpu.SemaphoreType.DMA((N_HOPS,)),
            pltpu.SemaphoreType.DMA((N_HOPS,)),
            pltpu.SemaphoreType.REGULAR,
        ],
        compiler_params=pltpu.CompilerParams(collective_id=0),
    )(x, Win0, Wout0, Win1, Wout1, Win2, Wout2)


# device time: 60159 ns/iter; 5.8658x vs baseline; 5.8658x over previous
import jax
import jax.numpy as jnp
from jax import lax
from jax.experimental import pallas as pl
from jax.experimental.pallas import tpu as pltpu

N_DEV = 16
N_LAYERS = 3


def kernel(x, Win0, Wout0, Win1, Wout1, Win2, Wout2):
    b, d = x.shape
    rows = b // N_DEV

    def body(x_ref, win0_ref, wout0_ref, win1_ref, wout1_ref, win2_ref,
             wout2_ref, out_ref, part_buf, rs_buf, x_buf,
             rs_send_sems, rs_recv_sems, ag_send_sems, ag_recv_sems):
        my = lax.axis_index("i")

        barrier_sem = pltpu.get_barrier_semaphore()
        for o in range(1, N_DEV):
            peer = lax.rem(my + o, N_DEV)
            pl.semaphore_signal(
                barrier_sem, inc=1,
                device_id=(peer,), device_id_type=pl.DeviceIdType.MESH,
            )
        pl.semaphore_wait(barrier_sem, N_DEV - 1)

        wins = (win0_ref, win1_ref, win2_ref)
        wouts = (wout0_ref, wout1_ref, wout2_ref)

        x_cur = x_ref[...]
        for layer in range(N_LAYERS):
            h = jnp.maximum(
                jnp.dot(x_cur, wins[layer][...],
                        preferred_element_type=jnp.float32),
                0.0,
            )
            partial = jnp.dot(h, wouts[layer][...],
                              preferred_element_type=jnp.float32)
            part_buf[...] = partial.reshape(N_DEV, rows, d)

            rs_sends = []
            for o in range(1, N_DEV):
                peer = lax.rem(my + o, N_DEV)
                rdma = pltpu.make_async_remote_copy(
                    src_ref=part_buf.at[peer],
                    dst_ref=rs_buf.at[my],
                    send_sem=rs_send_sems.at[peer],
                    recv_sem=rs_recv_sems.at[my],
                    device_id=(peer,),
                    device_id_type=pl.DeviceIdType.MESH,
                )
                rdma.start()
                rs_sends.append(rdma)
            rs_buf[my] = part_buf[my]

            for o in range(1, N_DEV):
                src = lax.rem(my + o, N_DEV)
                recv = pltpu.make_async_remote_copy(
                    src_ref=part_buf.at[0],
                    dst_ref=rs_buf.at[src],
                    send_sem=rs_send_sems.at[0],
                    recv_sem=rs_recv_sems.at[src],
                    device_id=(src,),
                    device_id_type=pl.DeviceIdType.MESH,
                )
                recv.wait_recv()
            acc = jnp.sum(rs_buf[...], axis=0)

            for rdma in rs_sends:
                rdma.wait_send()

            x_buf[my] = acc
            ag_sends = []
            for o in range(1, N_DEV):
                peer = lax.rem(my + o, N_DEV)
                rdma = pltpu.make_async_remote_copy(
                    src_ref=x_buf.at[my],
                    dst_ref=x_buf.at[my],
                    send_sem=ag_send_sems.at[peer],
                    recv_sem=ag_recv_sems.at[my],
                    device_id=(peer,),
                    device_id_type=pl.DeviceIdType.MESH,
                )
                rdma.start()
                ag_sends.append(rdma)

            for o in range(1, N_DEV):
                src = lax.rem(my + o, N_DEV)
                recv = pltpu.make_async_remote_copy(
                    src_ref=part_buf.at[0],
                    dst_ref=x_buf.at[src],
                    send_sem=ag_send_sems.at[0],
                    recv_sem=ag_recv_sems.at[src],
                    device_id=(src,),
                    device_id_type=pl.DeviceIdType.MESH,
                )
                recv.wait_recv()
            x_cur = x_buf[...].reshape(b, d)

            for rdma in ag_sends:
                rdma.wait_send()

        out_ref[...] = x_cur

    return pl.pallas_call(
        body,
        out_shape=jax.ShapeDtypeStruct((b, d), jnp.float32),
        in_specs=[pl.BlockSpec(memory_space=pltpu.VMEM)] * 7,
        out_specs=pl.BlockSpec(memory_space=pltpu.VMEM),
        scratch_shapes=[
            pltpu.VMEM((N_DEV, rows, d), jnp.float32),
            pltpu.VMEM((N_DEV, rows, d), jnp.float32),
            pltpu.VMEM((N_DEV, rows, d), jnp.float32),
            pltpu.SemaphoreType.DMA((N_DEV,)),
            pltpu.SemaphoreType.DMA((N_DEV,)),
            pltpu.SemaphoreType.DMA((N_DEV,)),
            pltpu.SemaphoreType.DMA((N_DEV,)),
        ],
        compiler_params=pltpu.CompilerParams(collective_id=0),
    )(x, Win0, Wout0, Win1, Wout1, Win2, Wout2)


# device time: 57303 ns/iter; 6.1582x vs baseline; 1.0498x over previous
import jax
import jax.numpy as jnp
from jax import lax
from jax.experimental import pallas as pl
from jax.experimental.pallas import tpu as pltpu

N_DEV = 16
N_LAYERS = 3


def kernel(x, Win0, Wout0, Win1, Wout1, Win2, Wout2):
    b, d = x.shape
    rows = b // N_DEV

    def body(x_ref, win0_ref, wout0_ref, win1_ref, wout1_ref, win2_ref,
             wout2_ref, out_ref, part_buf, rs_buf, x_buf,
             rs_send_sems, rs_recv_sems, ag_send_sems, ag_recv_sems):
        my = lax.axis_index("i")

        def rs_send(peer):
            rdma = pltpu.make_async_remote_copy(
                src_ref=part_buf.at[peer],
                dst_ref=rs_buf.at[my],
                send_sem=rs_send_sems.at[peer],
                recv_sem=rs_recv_sems.at[my],
                device_id=(peer,),
                device_id_type=pl.DeviceIdType.MESH,
            )
            rdma.start()
            return rdma

        def rs_wait_recv(src):
            pltpu.make_async_remote_copy(
                src_ref=part_buf.at[0], dst_ref=rs_buf.at[src],
                send_sem=rs_send_sems.at[0], recv_sem=rs_recv_sems.at[src],
                device_id=(src,), device_id_type=pl.DeviceIdType.MESH,
            ).wait_recv()

        def ag_send(peer):
            rdma = pltpu.make_async_remote_copy(
                src_ref=x_buf.at[my],
                dst_ref=x_buf.at[my],
                send_sem=ag_send_sems.at[peer],
                recv_sem=ag_recv_sems.at[my],
                device_id=(peer,),
                device_id_type=pl.DeviceIdType.MESH,
            )
            rdma.start()
            return rdma

        def ag_wait_recv(src):
            pltpu.make_async_remote_copy(
                src_ref=part_buf.at[0], dst_ref=x_buf.at[src],
                send_sem=ag_send_sems.at[0], recv_sem=ag_recv_sems.at[src],
                device_id=(src,), device_id_type=pl.DeviceIdType.MESH,
            ).wait_recv()

        barrier_sem = pltpu.get_barrier_semaphore()
        for o in range(1, N_DEV):
            peer = lax.rem(my + o, N_DEV)
            pl.semaphore_signal(
                barrier_sem, inc=1,
                device_id=(peer,), device_id_type=pl.DeviceIdType.MESH,
            )
        pl.semaphore_wait(barrier_sem, N_DEV - 1)

        wins = (win0_ref, win1_ref, win2_ref)
        wouts = (wout0_ref, wout1_ref, wout2_ref)

        h = jnp.maximum(
            jnp.dot(x_ref[...], wins[0][...],
                    preferred_element_type=jnp.float32),
            0.0,
        )
        partial = jnp.dot(h, wouts[0][...], preferred_element_type=jnp.float32)
        part_buf[...] = partial.reshape(N_DEV, rows, d)

        rs_sends = [rs_send(lax.rem(my + o, N_DEV)) for o in range(1, N_DEV)]
        rs_buf[my] = part_buf[my]
        for o in range(1, N_DEV):
            rs_wait_recv(lax.rem(my + o, N_DEV))
        acc = jnp.sum(rs_buf[...], axis=0)
        for rdma in rs_sends:
            rdma.wait_send()
        x_buf[my] = acc
        ag_sends = [ag_send(lax.rem(my + o, N_DEV)) for o in range(1, N_DEV)]

        for layer in range(1, N_LAYERS):
            win = wins[layer][...]
            wout = wouts[layer][...]

            h_my = jnp.maximum(
                jnp.dot(x_buf[my], win, preferred_element_type=jnp.float32),
                0.0,
            )
            rs_buf[my] = jnp.dot(h_my, wout, preferred_element_type=jnp.float32)

            rs_sends = []
            for o in range(1, N_DEV):
                p = lax.rem(my + o, N_DEV)
                ag_wait_recv(p)
                h_p = jnp.maximum(
                    jnp.dot(x_buf[p], win, preferred_element_type=jnp.float32),
                    0.0,
                )
                part_buf[p] = jnp.dot(h_p, wout,
                                      preferred_element_type=jnp.float32)
                rs_sends.append(rs_send(p))

            for rdma in ag_sends:
                rdma.wait_send()

            for o in range(1, N_DEV):
                rs_wait_recv(lax.rem(my + o, N_DEV))
            acc = jnp.sum(rs_buf[...], axis=0)
            for rdma in rs_sends:
                rdma.wait_send()
            x_buf[my] = acc
            ag_sends = [ag_send(lax.rem(my + o, N_DEV))
                        for o in range(1, N_DEV)]

        for o in range(1, N_DEV):
            ag_wait_recv(lax.rem(my + o, N_DEV))
        out_ref[...] = x_buf[...].reshape(b, d)
        for rdma in ag_sends:
            rdma.wait_send()

    return pl.pallas_call(
        body,
        out_shape=jax.ShapeDtypeStruct((b, d), jnp.float32),
        in_specs=[pl.BlockSpec(memory_space=pltpu.VMEM)] * 7,
        out_specs=pl.BlockSpec(memory_space=pltpu.VMEM),
        scratch_shapes=[
            pltpu.VMEM((N_DEV, rows, d), jnp.float32),
            pltpu.VMEM((N_DEV, rows, d), jnp.float32),
            pltpu.VMEM((N_DEV, rows, d), jnp.float32),
            pltpu.SemaphoreType.DMA((N_DEV,)),
            pltpu.SemaphoreType.DMA((N_DEV,)),
            pltpu.SemaphoreType.DMA((N_DEV,)),
            pltpu.SemaphoreType.DMA((N_DEV,)),
        ],
        compiler_params=pltpu.CompilerParams(collective_id=0),
    )(x, Win0, Wout0, Win1, Wout1, Win2, Wout2)


# device time: 47910 ns/iter; 7.3655x vs baseline; 1.1961x over previous
import jax
import jax.numpy as jnp
from jax import lax
from jax.experimental import pallas as pl
from jax.experimental.pallas import tpu as pltpu

N_DEV = 16
N_LAYERS = 3
G = 4
N_GROUPS = N_DEV // G

_FAR_FIRST = sorted(range(1, N_DEV), key=lambda o: -min(o, N_DEV - o))


def kernel(x, Win0, Wout0, Win1, Wout1, Win2, Wout2):
    b, d = x.shape
    rows = b // N_DEV
    grows = rows * G
    wire_dt = jnp.bfloat16

    def body(x_ref, win0_ref, wout0_ref, win1_ref, wout1_ref, win2_ref,
             wout2_ref, out_ref, part_buf, rs_buf, x_buf,
             rs_send_sems, rs_recv_sems, ag_send_sems, ag_recv_sems):
        my = lax.axis_index("i")

        def rs_send(s):
            pltpu.make_async_remote_copy(
                src_ref=part_buf.at[s],
                dst_ref=rs_buf.at[my],
                send_sem=rs_send_sems.at[s],
                recv_sem=rs_recv_sems.at[my],
                device_id=(s,),
                device_id_type=pl.DeviceIdType.MESH,
            ).start()

        def rs_wait_send(s):
            pltpu.make_async_remote_copy(
                src_ref=part_buf.at[s], dst_ref=rs_buf.at[my],
                send_sem=rs_send_sems.at[s], recv_sem=rs_recv_sems.at[my],
                device_id=(s,), device_id_type=pl.DeviceIdType.MESH,
            ).wait_send()

        def rs_wait_recv(src):
            pltpu.make_async_remote_copy(
                src_ref=part_buf.at[0], dst_ref=rs_buf.at[src],
                send_sem=rs_send_sems.at[0], recv_sem=rs_recv_sems.at[src],
                device_id=(src,), device_id_type=pl.DeviceIdType.MESH,
            ).wait_recv()

        def ag_send(peer):
            pltpu.make_async_remote_copy(
                src_ref=x_buf.at[my],
                dst_ref=x_buf.at[my],
                send_sem=ag_send_sems.at[peer],
                recv_sem=ag_recv_sems.at[my],
                device_id=(peer,),
                device_id_type=pl.DeviceIdType.MESH,
            ).start()

        def ag_wait_send(peer):
            pltpu.make_async_remote_copy(
                src_ref=x_buf.at[my], dst_ref=x_buf.at[my],
                send_sem=ag_send_sems.at[peer], recv_sem=ag_recv_sems.at[my],
                device_id=(peer,), device_id_type=pl.DeviceIdType.MESH,
            ).wait_send()

        def ag_wait_recv(src):
            pltpu.make_async_remote_copy(
                src_ref=part_buf.at[0], dst_ref=x_buf.at[src],
                send_sem=ag_send_sems.at[0], recv_sem=ag_recv_sems.at[src],
                device_id=(src,), device_id_type=pl.DeviceIdType.MESH,
            ).wait_recv()

        barrier_sem = pltpu.get_barrier_semaphore()
        for o in range(1, N_DEV):
            peer = lax.rem(my + o, N_DEV)
            pl.semaphore_signal(
                barrier_sem, inc=1,
                device_id=(peer,), device_id_type=pl.DeviceIdType.MESH,
            )
        pl.semaphore_wait(barrier_sem, N_DEV - 1)

        wins = (win0_ref, win1_ref, win2_ref)
        wouts = (wout0_ref, wout1_ref, wout2_ref)

        def compute_group(xg, layer):
            h = jnp.maximum(
                jnp.dot(xg, wins[layer][...],
                        preferred_element_type=jnp.float32),
                0.0,
            )
            return jnp.dot(h, wouts[layer][...],
                           preferred_element_type=jnp.float32)

        def send_group(g):
            for s in range(G * g, G * g + G):
                @pl.when(s != my)
                def _(s=s):
                    rs_send(s)

        for g in range(N_GROUPS):
            pg = compute_group(x_ref[pl.ds(g * grows, grows), :], 0)
            part_buf[pl.ds(G * g, G)] = pg.astype(wire_dt).reshape(G, rows, d)
            send_group(g)

        def finish_layer(prev_ag_inflight):
            rs_buf[my] = part_buf[my]
            for o in range(1, N_DEV):
                rs_wait_recv(lax.rem(my + o, N_DEV))
            acc = jnp.sum(rs_buf[...].astype(jnp.float32), axis=0)
            if prev_ag_inflight:
                for o in range(1, N_DEV):
                    ag_wait_send(lax.rem(my + o, N_DEV))
            x_buf[my] = acc.astype(wire_dt)
            for o in _FAR_FIRST:
                ag_send(lax.rem(my + o, N_DEV))
            for o in range(1, N_DEV):
                rs_wait_send(lax.rem(my + o, N_DEV))

        finish_layer(prev_ag_inflight=False)

        for layer in range(1, N_LAYERS):
            for g in range(N_GROUPS):
                for s in range(G * g, G * g + G):
                    @pl.when(s != my)
                    def _(s=s):
                        ag_wait_recv(s)
                xg = x_buf[pl.ds(G * g, G)].reshape(grows, d)
                pg = compute_group(xg.astype(jnp.float32), layer)
                part_buf[pl.ds(G * g, G)] = pg.astype(wire_dt).reshape(
                    G, rows, d)
                send_group(g)
            finish_layer(prev_ag_inflight=True)

        for o in range(1, N_DEV):
            ag_wait_recv(lax.rem(my + o, N_DEV))
        out_ref[...] = x_buf[...].reshape(b, d).astype(jnp.float32)
        for o in range(1, N_DEV):
            ag_wait_send(lax.rem(my + o, N_DEV))

    return pl.pallas_call(
        body,
        out_shape=jax.ShapeDtypeStruct((b, d), jnp.float32),
        in_specs=[pl.BlockSpec(memory_space=pltpu.VMEM)] * 7,
        out_specs=pl.BlockSpec(memory_space=pltpu.VMEM),
        scratch_shapes=[
            pltpu.VMEM((N_DEV, rows, d), wire_dt),
            pltpu.VMEM((N_DEV, rows, d), wire_dt),
            pltpu.VMEM((N_DEV, rows, d), wire_dt),
            pltpu.SemaphoreType.DMA((N_DEV,)),
            pltpu.SemaphoreType.DMA((N_DEV,)),
            pltpu.SemaphoreType.DMA((N_DEV,)),
            pltpu.SemaphoreType.DMA((N_DEV,)),
        ],
        compiler_params=pltpu.CompilerParams(collective_id=0),
    )(x, Win0, Wout0, Win1, Wout1, Win2, Wout2)
